# Initial kernel scaffold; baseline (speedup 1.0000x reference)
import jax
import jax.numpy as jnp
from jax.experimental import pallas as pl


def kernel(x):
    raise NotImplementedError("write your pallas kernel here")


# speedup vs baseline: 37.2477x; 37.2477x over previous
import jax
import jax.numpy as jnp
from jax import lax
from jax.experimental import pallas as pl
from jax.experimental.pallas import tpu as pltpu
from jax.experimental.pallas import tpu_sc as plsc

_B, _C, _H, _W = 4, 3, 512, 512
_P, _S = 16, 8
_NH = (_H - _P) // _S + 1
_NW = (_W - _P) // _S + 1
_N = _NH * _NW
_F = _C * _P * _P
_K = 8
_NWORK = 32
_ROWS = _B * _NH
_NE = _N * _K
_EQ = 8064
_EV = 7560
_BIG = 2**31 - 1

_OFFS = [(di, dj) for di in range(-2, 3) for dj in range(-2, 3)
         if (di, dj) != (0, 0)]


def _sc_body(x_hbm, patches_hbm, pos_hbm, edges_hbm, in_v, out_v, edg_v, pos_v):
    wid = lax.axis_index("s") * 2 + lax.axis_index("c")
    lane = lax.iota(jnp.int32, 16)

    def do_row(r):
        b = lax.div(r, _NH)
        i = lax.rem(r, _NH)
        for c in range(_C):
            pltpu.sync_copy(
                x_hbm.at[pl.ds(((b * _C + c) * _H + i * _S) * _W, _P * _W)],
                in_v.at[pl.ds(c * _P * _W, _P * _W)])

        def jbody(j, carry):
            jw = j * _S
            for c in range(_C):
                for kh in range(_P):
                    v = plsc.load_gather(
                        in_v, [jw + (c * _P * _W + kh * _W) + lane])
                    out_v[pl.ds(j * _F + c * _P * _P + kh * _P, _P)] = v
            return carry

        lax.fori_loop(0, _NW, jbody, 0)
        pltpu.sync_copy(out_v,
                        patches_hbm.at[pl.ds((b * _N + i * _NW) * _F, _NW * _F)])

    for t in range(7):
        do_row(wid + _NWORK * t)

    @pl.when(wid < _ROWS - 7 * _NWORK)
    def _():
        do_row(wid + 7 * _NWORK)

    @pl.when(wid == 27)
    def _():
        def pbody(ch, carry):
            n = ch * 16 + lane
            gi = lax.div(n, _NW)
            gj = lax.rem(n, _NW)
            plsc.store_scatter(pos_v, [n * 2], gi.astype(jnp.float32))
            plsc.store_scatter(pos_v, [n * 2 + 1], gj.astype(jnp.float32))
            return carry

        lax.fori_loop(0, 256, pbody, 0)
        pltpu.sync_copy(pos_v, pos_hbm)

    @pl.when(wid >= 28)
    def _():
        q = wid - 28

        def ebody(ch, carry):
            n = q * 1008 + ch * 16 + lane
            validn = n < _N
            gi = lax.div(n, _NW)
            gj = lax.rem(n, _NW)
            keys = []
            for di in range(-2, 3):
                ii = gi + di
                vi = (ii >= 0) & (ii < _NH) & validn
                for dj in range(-2, 3):
                    if (di, dj) == (0, 0):
                        continue
                    jj = gj + dj
                    v = vi & (jj >= 0) & (jj < _NW)
                    d2 = di * di + dj * dj
                    keys.append(jnp.where(v, n + (d2 * 4096 + di * _NW + dj),
                                          _BIG))
            floc = (ch * 16 + lane) * _K
            for k in range(_K):
                m = keys[0]
                for kk in keys[1:]:
                    m = jnp.minimum(m, kk)
                nbr = jnp.bitwise_and(m, 4095)
                plsc.store_scatter(edg_v, [floc + k], n)
                plsc.store_scatter(edg_v, [_EQ + floc + k], nbr)
                keys = [jnp.where(kk == m, _BIG, kk) for kk in keys]
            return carry

        lax.fori_loop(0, 63, ebody, 0)
        pltpu.sync_copy(edg_v.at[pl.ds(0, _EV)],
                        edges_hbm.at[pl.ds(q * _EQ, _EV)])
        pltpu.sync_copy(edg_v.at[pl.ds(_EQ, _EV)],
                        edges_hbm.at[pl.ds(_NE + q * _EQ, _EV)])

        @pl.when(q < 3)
        def _():
            pltpu.sync_copy(edg_v.at[pl.ds(_EV, _EQ - _EV)],
                            edges_hbm.at[pl.ds(q * _EQ + _EV, _EQ - _EV)])
            pltpu.sync_copy(edg_v.at[pl.ds(_EQ + _EV, _EQ - _EV)],
                            edges_hbm.at[pl.ds(_NE + q * _EQ + _EV, _EQ - _EV)])


def kernel(x):
    mesh = plsc.VectorSubcoreMesh(core_axis_name="c", subcore_axis_name="s")
    patches_f, pos_f, edges_f = pl.kernel(
        _sc_body,
        out_type=[
            jax.ShapeDtypeStruct((_B * _N * _F,), jnp.float32),
            jax.ShapeDtypeStruct((8192,), jnp.float32),
            jax.ShapeDtypeStruct((2 * _NE,), jnp.int32),
        ],
        mesh=mesh,
        compiler_params=pltpu.CompilerParams(needs_layout_passes=False),
        scratch_types=[
            pltpu.VMEM((_C * _P * _W,), jnp.float32),
            pltpu.VMEM((_NW * _F,), jnp.float32),
            pltpu.VMEM((2 * _EQ,), jnp.int32),
            pltpu.VMEM((8192,), jnp.float32),
        ],
    )(x.reshape(-1))
    patches = patches_f.reshape(_B, _N, _F)
    pos = pos_f.reshape(4096, 2)[:_N]
    positions_b = jnp.broadcast_to(pos[None], (_B, _N, 2))
    edge_index_all = jnp.broadcast_to(edges_f.reshape(2, _NE)[None],
                                      (_B, 2, _NE))
    return patches, positions_b, edge_index_all


# speedup vs baseline: 40.6076x; 1.0902x over previous
import jax
import jax.numpy as jnp
from jax import lax
from jax.experimental import pallas as pl
from jax.experimental.pallas import tpu as pltpu
from jax.experimental.pallas import tpu_sc as plsc

_B, _C, _H, _W = 4, 3, 512, 512
_P, _S = 16, 8
_NH = (_H - _P) // _S + 1
_NW = (_W - _P) // _S + 1
_N = _NH * _NW
_F = _C * _P * _P
_K = 8
_NWORK = 32
_ROWS = _B * _NH
_NE = _N * _K
_EQ = 8064
_EV = 7560
_BIG = 2**31 - 1
_SLAB = _C * _P * _W
_HALF = 32 * _F

_OFFS = [(di, dj) for di in range(-2, 3) for dj in range(-2, 3)
         if (di, dj) != (0, 0)]


def _sc_body(x_hbm, patches_hbm, pos_hbm, edges_hbm, in_v, out_v, edg_v, pos_v,
             sem_in, sem_out0, sem_out1):
    wid = lax.axis_index("s") * 2 + lax.axis_index("c")
    lane = lax.iota(jnp.int32, 16)
    sems_out = (sem_out0, sem_out1)

    def row_bi(t):
        r = jnp.minimum(wid + _NWORK * t, _ROWS - 1)
        return lax.div(r, _NH), lax.rem(r, _NH)

    def valid(t):
        return (wid + _NWORK * t) < _ROWS

    def in_copies(t):
        b, i = row_bi(t)
        buf = jnp.bitwise_and(t, 1)
        return [pltpu.make_async_copy(
            x_hbm.at[pl.ds(((b * _C + c) * _H + i * _S) * _W, _P * _W)],
            in_v.at[pl.ds(buf * _SLAB + c * _P * _W, _P * _W)],
            sem_in) for c in range(_C)]

    def out_copy(t, h):
        b, i = row_bi(t)
        ln = 32 if h == 0 else 31
        return pltpu.make_async_copy(
            out_v.at[pl.ds(h * _HALF, ln * _F)],
            patches_hbm.at[pl.ds((b * _N + i * _NW + h * 32) * _F, ln * _F)],
            sems_out[h])

    def compute_half(t, h):
        ln = 32 if h == 0 else 31
        base = jnp.bitwise_and(t, 1) * _SLAB

        def jbody(jl, carry):
            jw = (jl + h * 32) * _S + base
            for c in range(_C):
                for kh in range(_P):
                    v = plsc.load_gather(
                        in_v, [jw + (c * _P * _W + kh * _W) + lane])
                    out_v[pl.ds(h * _HALF + jl * _F + c * _P * _P + kh * _P,
                                _P)] = v
            return carry

        lax.fori_loop(0, ln, jbody, 0)

    for cp in in_copies(0):
        cp.start()

    def tbody(t, carry):
        @pl.when(valid(t))
        def _():
            for cp in in_copies(t):
                cp.wait()

            @pl.when(valid(t + 1))
            def _():
                for cp in in_copies(t + 1):
                    cp.start()

            for h in (0, 1):
                @pl.when(t > 0)
                def _():
                    out_copy(t - 1, h).wait()

                compute_half(t, h)
                out_copy(t, h).start()
        return carry

    lax.fori_loop(0, 8, tbody, 0)

    @pl.when(wid < _ROWS - 7 * _NWORK)
    def _():
        out_copy(7, 0).wait()
        out_copy(7, 1).wait()

    @pl.when(wid == 27)
    def _():
        def pbody(ch, carry):
            n = ch * 16 + lane
            gi = lax.div(n, _NW)
            gj = lax.rem(n, _NW)
            plsc.store_scatter(pos_v, [n * 2], gi.astype(jnp.float32))
            plsc.store_scatter(pos_v, [n * 2 + 1], gj.astype(jnp.float32))
            return carry

        lax.fori_loop(0, 256, pbody, 0)
        pltpu.sync_copy(pos_v, pos_hbm)

    @pl.when(wid >= _ROWS - 7 * _NWORK)
    def _():
        out_copy(6, 0).wait()
        out_copy(6, 1).wait()
        q = wid - 28

        def ebody(ch, carry):
            n = q * 1008 + ch * 16 + lane
            validn = n < _N
            gi = lax.div(n, _NW)
            gj = lax.rem(n, _NW)
            keys = []
            for di in range(-2, 3):
                ii = gi + di
                vi = (ii >= 0) & (ii < _NH) & validn
                for dj in range(-2, 3):
                    if (di, dj) == (0, 0):
                        continue
                    jj = gj + dj
                    v = vi & (jj >= 0) & (jj < _NW)
                    d2 = di * di + dj * dj
                    keys.append(jnp.where(v, n + (d2 * 4096 + di * _NW + dj),
                                          _BIG))
            floc = (ch * 16 + lane) * _K
            for k in range(_K):
                m = keys[0]
                for kk in keys[1:]:
                    m = jnp.minimum(m, kk)
                nbr = jnp.bitwise_and(m, 4095)
                plsc.store_scatter(edg_v, [floc + k], n)
                plsc.store_scatter(edg_v, [_EQ + floc + k], nbr)
                keys = [jnp.where(kk == m, _BIG, kk) for kk in keys]
            return carry

        lax.fori_loop(0, 63, ebody, 0)
        pltpu.sync_copy(edg_v.at[pl.ds(0, _EV)],
                        edges_hbm.at[pl.ds(q * _EQ, _EV)])
        pltpu.sync_copy(edg_v.at[pl.ds(_EQ, _EV)],
                        edges_hbm.at[pl.ds(_NE + q * _EQ, _EV)])

        @pl.when(q < 3)
        def _():
            pltpu.sync_copy(edg_v.at[pl.ds(_EV, _EQ - _EV)],
                            edges_hbm.at[pl.ds(q * _EQ + _EV, _EQ - _EV)])
            pltpu.sync_copy(edg_v.at[pl.ds(_EQ + _EV, _EQ - _EV)],
                            edges_hbm.at[pl.ds(_NE + q * _EQ + _EV, _EQ - _EV)])


def kernel(x):
    mesh = plsc.VectorSubcoreMesh(core_axis_name="c", subcore_axis_name="s")
    patches_f, pos_f, edges_f = pl.kernel(
        _sc_body,
        out_type=[
            jax.ShapeDtypeStruct((_B * _N * _F,), jnp.float32),
            jax.ShapeDtypeStruct((8192,), jnp.float32),
            jax.ShapeDtypeStruct((2 * _NE,), jnp.int32),
        ],
        mesh=mesh,
        compiler_params=pltpu.CompilerParams(needs_layout_passes=False),
        scratch_types=[
            pltpu.VMEM((2 * _SLAB,), jnp.float32),
            pltpu.VMEM((2 * _HALF,), jnp.float32),
            pltpu.VMEM((2 * _EQ,), jnp.int32),
            pltpu.VMEM((8192,), jnp.float32),
            pltpu.SemaphoreType.DMA,
            pltpu.SemaphoreType.DMA,
            pltpu.SemaphoreType.DMA,
        ],
    )(x.reshape(-1))
    patches = patches_f.reshape(_B, _N, _F)
    pos = pos_f.reshape(4096, 2)[:_N]
    positions_b = jnp.broadcast_to(pos[None], (_B, _N, 2))
    edge_index_all = jnp.broadcast_to(edges_f.reshape(2, _NE)[None],
                                      (_B, 2, _NE))
    return patches, positions_b, edge_index_all


# speedup vs baseline: 47.1633x; 1.1614x over previous
import jax
import jax.numpy as jnp
from jax import lax
from jax.experimental import pallas as pl
from jax.experimental.pallas import tpu as pltpu
from jax.experimental.pallas import tpu_sc as plsc

_B, _C, _H, _W = 4, 3, 512, 512
_P, _S = 16, 8
_NH = (_H - _P) // _S + 1
_NW = (_W - _P) // _S + 1
_N = _NH * _NW
_F = _C * _P * _P
_K = 8
_NWORK = 32
_ROWS = _B * _NH
_NE = _N * _K
_NCH = (_N + 15) // 16
_BIG = 2**31 - 1
_SLAB = _C * _P * _W
_HALF = 32 * _F


def _tree_min(vs):
    while len(vs) > 1:
        nxt = [jnp.minimum(vs[i], vs[i + 1]) for i in range(0, len(vs) - 1, 2)]
        if len(vs) % 2:
            nxt.append(vs[-1])
        vs = nxt
    return vs[0]


def _sc_body(x_hbm, patches_hbm, pos_hbm, edges_hbm, in_v, out_v, edg_v,
             sem_in, sem_out0, sem_out1, sem_e):
    wid = lax.axis_index("s") * 2 + lax.axis_index("c")
    lane = lax.iota(jnp.int32, 16)
    sems_out = (sem_out0, sem_out1)

    def row_bi(t):
        r = jnp.minimum(wid + _NWORK * t, _ROWS - 1)
        return lax.div(r, _NH), lax.rem(r, _NH)

    def valid(t):
        return (wid + _NWORK * t) < _ROWS

    def in_copies(t):
        b, i = row_bi(t)
        buf = jnp.bitwise_and(t, 1)
        return [pltpu.make_async_copy(
            x_hbm.at[pl.ds(((b * _C + c) * _H + i * _S) * _W, _P * _W)],
            in_v.at[pl.ds(buf * _SLAB + c * _P * _W, _P * _W)],
            sem_in) for c in range(_C)]

    def out_copy(t, h):
        b, i = row_bi(t)
        ln = 32 if h == 0 else 31
        return pltpu.make_async_copy(
            out_v.at[pl.ds(h * _HALF, ln * _F)],
            patches_hbm.at[pl.ds((b * _N + i * _NW + h * 32) * _F, ln * _F)],
            sems_out[h])

    def compute_half(t, h):
        ln = 32 if h == 0 else 31
        base = jnp.bitwise_and(t, 1) * _SLAB

        @plsc.parallel_loop(0, ln, unroll=2)
        def _(jl):
            jw = (jl + h * 32) * _S + base
            for c in range(_C):
                for kh in range(_P):
                    v = plsc.load_gather(
                        in_v, [jw + (c * _P * _W + kh * _W) + lane])
                    out_v[pl.ds(h * _HALF + jl * _F + c * _P * _P + kh * _P,
                                _P)] = v

    for cp in in_copies(0):
        cp.start()

    nch = jnp.minimum(8, _NCH - wid * 8)

    @plsc.parallel_loop(0, nch)
    def _(chl):
        n = (wid * 8 + chl) * 16 + lane
        validn = n < _N
        gi = lax.div(n, _NW)
        gj = lax.rem(n, _NW)
        keys = []
        for di in range(-2, 3):
            ii = gi + di
            vi = (ii >= 0) & (ii < _NH) & validn
            for dj in range(-2, 3):
                if (di, dj) == (0, 0):
                    continue
                jj = gj + dj
                v = vi & (jj >= 0) & (jj < _NW)
                d2 = di * di + dj * dj
                keys.append(jnp.where(v, n + (d2 * 4096 + di * _NW + dj),
                                      _BIG))
        floc = chl * 128 + lane * _K
        for k in range(_K):
            m = _tree_min(keys)
            nbr = jnp.bitwise_and(m, 4095)
            plsc.store_scatter(edg_v, [floc + k], n)
            plsc.store_scatter(edg_v, [1024 + floc + k], nbr)
            if k < _K - 1:
                keys = [jnp.where(kk == m, _BIG, kk) for kk in keys]

    def edge_out(ln):
        for b in range(_B):
            for row in (0, 1):
                pltpu.make_async_copy(
                    edg_v.at[pl.ds(row * 1024, ln)],
                    edges_hbm.at[pl.ds(b * 2 * _NE + row * _NE + wid * 1024,
                                       ln)],
                    sem_e).start()

    @pl.when(wid < _NWORK - 1)
    def _():
        edge_out(1024)

    @pl.when(wid == _NWORK - 1)
    def _():
        edge_out(8)

        @plsc.parallel_loop(0, _NCH)
        def _(ch):
            n = ch * 16 + lane
            validn = n < _N
            nc = jnp.where(validn, n, 0)
            gi = lax.div(nc, _NW).astype(jnp.float32)
            gj = lax.rem(nc, _NW).astype(jnp.float32)
            for b in range(_B):
                plsc.store_scatter(out_v, [b * (2 * _N) + nc * 2], gi,
                                   mask=validn)
                plsc.store_scatter(out_v, [b * (2 * _N) + nc * 2 + 1], gj,
                                   mask=validn)

        pltpu.sync_copy(out_v.at[pl.ds(0, _B * 2 * _N)], pos_hbm)

    def tbody(t, carry):
        @pl.when(valid(t))
        def _():
            for cp in in_copies(t):
                cp.wait()

            @pl.when(valid(t + 1))
            def _():
                for cp in in_copies(t + 1):
                    cp.start()

            for h in (0, 1):
                @pl.when(t > 0)
                def _():
                    out_copy(t - 1, h).wait()

                compute_half(t, h)
                out_copy(t, h).start()
        return carry

    lax.fori_loop(0, 8, tbody, 0)

    @pl.when(wid < _ROWS - 7 * _NWORK)
    def _():
        out_copy(7, 0).wait()
        out_copy(7, 1).wait()

    @pl.when(wid >= _ROWS - 7 * _NWORK)
    def _():
        out_copy(6, 0).wait()
        out_copy(6, 1).wait()

    def edge_drain(ln):
        for b in range(_B):
            for row in (0, 1):
                pltpu.make_async_copy(
                    edg_v.at[pl.ds(row * 1024, ln)],
                    edges_hbm.at[pl.ds(b * 2 * _NE + row * _NE + wid * 1024,
                                       ln)],
                    sem_e).wait()

    @pl.when(wid < _NWORK - 1)
    def _():
        edge_drain(1024)

    @pl.when(wid == _NWORK - 1)
    def _():
        edge_drain(8)


def kernel(x):
    mesh = plsc.VectorSubcoreMesh(core_axis_name="c", subcore_axis_name="s")
    patches_f, pos_f, edges_f = pl.kernel(
        _sc_body,
        out_type=[
            jax.ShapeDtypeStruct((_B * _N * _F,), jnp.float32),
            jax.ShapeDtypeStruct((_B * _N * 2,), jnp.float32),
            jax.ShapeDtypeStruct((_B * 2 * _NE,), jnp.int32),
        ],
        mesh=mesh,
        compiler_params=pltpu.CompilerParams(needs_layout_passes=False),
        scratch_types=[
            pltpu.VMEM((2 * _SLAB,), jnp.float32),
            pltpu.VMEM((2 * _HALF,), jnp.float32),
            pltpu.VMEM((2048,), jnp.int32),
            pltpu.SemaphoreType.DMA,
            pltpu.SemaphoreType.DMA,
            pltpu.SemaphoreType.DMA,
            pltpu.SemaphoreType.DMA,
        ],
    )(x.reshape(-1))
    return (patches_f.reshape(_B, _N, _F),
            pos_f.reshape(_B, _N, 2),
            edges_f.reshape(_B, 2, _NE))


# speedup vs baseline: 117.2109x; 2.4852x over previous
import jax
import jax.numpy as jnp
from jax import lax
from jax.experimental import pallas as pl
from jax.experimental.pallas import tpu as pltpu
from jax.experimental.pallas import tpu_sc as plsc

_B, _C, _H, _W = 4, 3, 512, 512
_P, _S = 16, 8
_NH = (_H - _P) // _S + 1
_NW = (_W - _P) // _S + 1
_N = _NH * _NW
_F = _C * _P * _P
_K = 8
_NWORK = 32
_ROWS = _B * _NH
_NE = _N * _K
_NCH = (_N + 15) // 16
_BIG = 2**31 - 1
_SLAB = _C * _P * _W
_HALF = 32 * _F


def _tree_min(vs):
    while len(vs) > 1:
        nxt = [jnp.minimum(vs[i], vs[i + 1]) for i in range(0, len(vs) - 1, 2)]
        if len(vs) % 2:
            nxt.append(vs[-1])
        vs = nxt
    return vs[0]


def _sc_body(x_hbm, patches_hbm, pos_hbm, edges_hbm, in_v, out_v, edg_v,
             sem_in, sem_out0, sem_out1, sem_e):
    wid = lax.axis_index("s") * 2 + lax.axis_index("c")
    lane = lax.iota(jnp.int32, 16)
    sems_out = (sem_out0, sem_out1)

    def row_bi(t):
        r = jnp.minimum(wid + _NWORK * t, _ROWS - 1)
        return lax.div(r, _NH), lax.rem(r, _NH)

    def valid(t):
        return (wid + _NWORK * t) < _ROWS

    def in_copies(t):
        b, i = row_bi(t)
        buf = jnp.bitwise_and(t, 1)
        return [pltpu.make_async_copy(
            x_hbm.at[pl.ds(((b * _C + c) * _H + i * _S) * _W, _P * _W)],
            in_v.at[pl.ds(buf * _SLAB + c * _P * _W, _P * _W)],
            sem_in) for c in range(_C)]

    def out_copies(t, h):
        b, i = row_bi(t)
        ln = 32 if h == 0 else 31
        return [pltpu.make_async_copy(
            out_v.at[pl.ds((h * 6 + tc) * 32, ln), :],
            patches_hbm.at[pl.ds(i * _NW + h * 32, ln), tc * _B + b, :],
            sems_out[h]) for tc in range(_F // 128)]

    def compute_half(t, h):
        ln = 32 if h == 0 else 31
        base = jnp.bitwise_and(t, 1) * _SLAB

        segs = [(c, kh) for c in range(_C) for kh in range(_P)]

        @plsc.parallel_loop(0, ln, unroll=2)
        def _(jl):
            jw = (jl + h * 32) * _S + base
            for g in range(0, len(segs), 8):
                grp = segs[g:g + 8]
                vs = [plsc.load_gather(
                    in_v, [jw + (c * _P * _W + kh * _W) + lane])
                    for c, kh in grp]
                for (c, kh), v in zip(grp, vs):
                    f = c * _P * _P + kh * _P
                    out_v[(h * 6 + f // 128) * 32 + jl,
                          pl.ds(f % 128, _P)] = v

    @pl.when(wid < _NWORK - 1)
    def _():
        for cp in in_copies(0):
            cp.start()

    nch = jnp.minimum(8, _NCH - wid * 8)

    @plsc.parallel_loop(0, nch)
    def _(chl):
        n = (wid * 8 + chl) * 16 + lane
        validn = n < _N
        gi = lax.div(n, _NW)
        gj = lax.rem(n, _NW)
        keys = []
        for di in range(-2, 3):
            ii = gi + di
            vi = (ii >= 0) & (ii < _NH) & validn
            for dj in range(-2, 3):
                if (di, dj) == (0, 0):
                    continue
                jj = gj + dj
                v = vi & (jj >= 0) & (jj < _NW)
                d2 = di * di + dj * dj
                keys.append(jnp.where(v, n + (d2 * 4096 + di * _NW + dj),
                                      _BIG))
        floc = chl * 128 + lane * _K
        for k in range(_K):
            m = _tree_min(keys)
            nbr = jnp.bitwise_and(m, 4095)
            plsc.store_scatter(edg_v, [floc + k], n)
            plsc.store_scatter(edg_v, [1024 + floc + k], nbr)
            if k < _K - 1:
                keys = [jnp.where(kk == m, _BIG, kk) for kk in keys]

    def edge_out(ln):
        for b in range(_B):
            for row in (0, 1):
                pltpu.make_async_copy(
                    edg_v.at[pl.ds(row * 1024, ln)],
                    edges_hbm.at[pl.ds(b * 2 * _NE + row * _NE + wid * 1024,
                                       ln)],
                    sem_e).start()

    @pl.when(wid < _NWORK - 1)
    def _():
        edge_out(1024)

    @pl.when(wid == _NWORK - 1)
    def _():
        edge_out(8)

        @plsc.parallel_loop(0, _NCH)
        def _(ch):
            n = ch * 16 + lane
            validn = n < _N
            nc = jnp.where(validn, n, 0)
            gi = lax.div(nc, _NW).astype(jnp.float32)
            gj = lax.rem(nc, _NW).astype(jnp.float32)
            for b in range(_B):
                plsc.store_scatter(in_v, [b * (2 * _N) + nc * 2], gi,
                                   mask=validn)
                plsc.store_scatter(in_v, [b * (2 * _N) + nc * 2 + 1], gj,
                                   mask=validn)

        pltpu.sync_copy(in_v.at[pl.ds(0, _B * 2 * _N)], pos_hbm)
        for cp in in_copies(0):
            cp.start()

    def tbody(t, carry):
        @pl.when(valid(t))
        def _():
            for cp in in_copies(t):
                cp.wait()

            @pl.when(valid(t + 1))
            def _():
                for cp in in_copies(t + 1):
                    cp.start()

            for h in (0, 1):
                @pl.when(t > 0)
                def _():
                    for cp in out_copies(t - 1, h):
                        cp.wait()

                compute_half(t, h)
                for cp in out_copies(t, h):
                    cp.start()
        return carry

    lax.fori_loop(0, 8, tbody, 0)

    @pl.when(wid < _ROWS - 7 * _NWORK)
    def _():
        for h in (0, 1):
            for cp in out_copies(7, h):
                cp.wait()

    @pl.when(wid >= _ROWS - 7 * _NWORK)
    def _():
        for h in (0, 1):
            for cp in out_copies(6, h):
                cp.wait()

    def edge_drain(ln):
        for b in range(_B):
            for row in (0, 1):
                pltpu.make_async_copy(
                    edg_v.at[pl.ds(row * 1024, ln)],
                    edges_hbm.at[pl.ds(b * 2 * _NE + row * _NE + wid * 1024,
                                       ln)],
                    sem_e).wait()

    @pl.when(wid < _NWORK - 1)
    def _():
        edge_drain(1024)

    @pl.when(wid == _NWORK - 1)
    def _():
        edge_drain(8)


def kernel(x):
    mesh = plsc.VectorSubcoreMesh(core_axis_name="c", subcore_axis_name="s")
    patches_f, pos_f, edges_f = pl.kernel(
        _sc_body,
        out_type=[
            jax.ShapeDtypeStruct((_N, _F // 128 * _B, 128), jnp.float32),
            jax.ShapeDtypeStruct((_B * _N * 2,), jnp.float32),
            jax.ShapeDtypeStruct((_B * 2 * _NE,), jnp.int32),
        ],
        mesh=mesh,
        compiler_params=pltpu.CompilerParams(needs_layout_passes=False),
        scratch_types=[
            pltpu.VMEM((2 * _SLAB,), jnp.float32),
            pltpu.VMEM((2 * _F // 128 * 32, 128), jnp.float32),
            pltpu.VMEM((2048,), jnp.int32),
            pltpu.SemaphoreType.DMA,
            pltpu.SemaphoreType.DMA,
            pltpu.SemaphoreType.DMA,
            pltpu.SemaphoreType.DMA,
        ],
    )(x.reshape(-1))
    return (jnp.transpose(patches_f.reshape(_N, _F // 128, _B, 128),
                          (2, 0, 1, 3)).reshape(_B, _N, _F),
            pos_f.reshape(_B, _N, 2),
            edges_f.reshape(_B, 2, _NE))


# speedup vs baseline: 126.8053x; 1.0819x over previous
import jax
import jax.numpy as jnp
from jax import lax
from jax.experimental import pallas as pl
from jax.experimental.pallas import tpu as pltpu
from jax.experimental.pallas import tpu_sc as plsc

_B, _C, _H, _W = 4, 3, 512, 512
_P, _S = 16, 8
_NH = (_H - _P) // _S + 1
_NW = (_W - _P) // _S + 1
_N = _NH * _NW
_F = _C * _P * _P
_K = 8
_NWORK = 32
_ROWS = _B * _NH
_NE = _N * _K
_NCH = (_N + 15) // 16
_BIG = 2**31 - 1
_SLAB = _C * _P * _W
_HALF = 32 * _F


def _tree_min(vs):
    while len(vs) > 1:
        nxt = [jnp.minimum(vs[i], vs[i + 1]) for i in range(0, len(vs) - 1, 2)]
        if len(vs) % 2:
            nxt.append(vs[-1])
        vs = nxt
    return vs[0]


def _sc_body(x_hbm, patches_hbm, pos_hbm, edges_hbm, in_v, out_v, edg_v,
             sem_in, sem_out0, sem_out1, sem_e):
    wid = lax.axis_index("s") * 2 + lax.axis_index("c")
    lane = lax.iota(jnp.int32, 16)
    sems_out = (sem_out0, sem_out1)

    def row_bi(t):
        r = jnp.minimum(wid + _NWORK * t, _ROWS - 1)
        return lax.div(r, _NH), lax.rem(r, _NH)

    def valid(t):
        return (wid + _NWORK * t) < _ROWS

    def in_copies(t):
        b, i = row_bi(t)
        buf = jnp.bitwise_and(t, 1)
        return [pltpu.make_async_copy(
            x_hbm.at[pl.ds(((b * _C + c) * (_H // 8) + i) * 4096, _P * _W)],
            in_v.at[pl.ds(buf * _SLAB + c * _P * _W, _P * _W)],
            sem_in) for c in range(_C)]

    def out_copies(t, h):
        b, i = row_bi(t)
        ln = 32 if h == 0 else 31
        return [pltpu.make_async_copy(
            out_v.at[pl.ds((h * 6 + tc) * 32, ln), :],
            patches_hbm.at[pl.ds(i * _NW + h * 32, ln), tc * _B + b, :],
            sems_out[h]) for tc in range(_F // 128)]

    def compute_half(t, h):
        ln = 32 if h == 0 else 31
        base = jnp.bitwise_and(t, 1) * _SLAB

        segs = [(c, kh) for c in range(_C) for kh in range(_P)]

        @plsc.parallel_loop(0, ln, unroll=2)
        def _(jl):
            col = (jl + h * 32) * _S + lane
            cadj = col + jnp.right_shift(col, 7) * 896 + base
            for g in range(0, len(segs), 8):
                grp = segs[g:g + 8]
                vs = [plsc.load_gather(
                    in_v, [cadj + (c * _P * _W + (kh // 8) * 4096
                                   + (kh % 8) * 128)])
                    for c, kh in grp]
                for (c, kh), v in zip(grp, vs):
                    f = c * _P * _P + kh * _P
                    out_v[(h * 6 + f // 128) * 32 + jl,
                          pl.ds(f % 128, _P)] = v

    @pl.when(wid < _NWORK - 1)
    def _():
        for cp in in_copies(0):
            cp.start()

    nch = jnp.minimum(8, _NCH - wid * 8)

    @plsc.parallel_loop(0, nch)
    def _(chl):
        n = (wid * 8 + chl) * 16 + lane
        validn = n < _N
        gi = lax.div(n, _NW)
        gj = lax.rem(n, _NW)
        keys = []
        for di in range(-2, 3):
            ii = gi + di
            vi = (ii >= 0) & (ii < _NH) & validn
            for dj in range(-2, 3):
                if (di, dj) == (0, 0):
                    continue
                jj = gj + dj
                v = vi & (jj >= 0) & (jj < _NW)
                d2 = di * di + dj * dj
                keys.append(jnp.where(v, n + (d2 * 4096 + di * _NW + dj),
                                      _BIG))
        floc = chl * 128 + lane * _K
        for k in range(_K):
            m = _tree_min(keys)
            nbr = jnp.bitwise_and(m, 4095)
            plsc.store_scatter(edg_v, [floc + k], n)
            plsc.store_scatter(edg_v, [1024 + floc + k], nbr)
            if k < _K - 1:
                keys = [jnp.where(kk == m, _BIG, kk) for kk in keys]

    def edge_out(ln):
        for b in range(_B):
            for row in (0, 1):
                pltpu.make_async_copy(
                    edg_v.at[pl.ds(row * 1024, ln)],
                    edges_hbm.at[pl.ds(b * 2 * _NE + row * _NE + wid * 1024,
                                       ln)],
                    sem_e).start()

    @pl.when(wid < _NWORK - 1)
    def _():
        edge_out(1024)

    @pl.when(wid == _NWORK - 1)
    def _():
        edge_out(8)

        @plsc.parallel_loop(0, _NCH)
        def _(ch):
            n = ch * 16 + lane
            validn = n < _N
            nc = jnp.where(validn, n, 0)
            gi = lax.div(nc, _NW).astype(jnp.float32)
            gj = lax.rem(nc, _NW).astype(jnp.float32)
            for b in range(_B):
                plsc.store_scatter(in_v, [b * (2 * _N) + nc * 2], gi,
                                   mask=validn)
                plsc.store_scatter(in_v, [b * (2 * _N) + nc * 2 + 1], gj,
                                   mask=validn)

        pltpu.sync_copy(in_v.at[pl.ds(0, _B * 2 * _N)], pos_hbm)
        for cp in in_copies(0):
            cp.start()

    def tbody(t, carry):
        @pl.when(valid(t))
        def _():
            for cp in in_copies(t):
                cp.wait()

            @pl.when(valid(t + 1))
            def _():
                for cp in in_copies(t + 1):
                    cp.start()

            for h in (0, 1):
                @pl.when(t > 0)
                def _():
                    for cp in out_copies(t - 1, h):
                        cp.wait()

                compute_half(t, h)
                for cp in out_copies(t, h):
                    cp.start()
        return carry

    lax.fori_loop(0, 8, tbody, 0)

    @pl.when(wid < _ROWS - 7 * _NWORK)
    def _():
        for h in (0, 1):
            for cp in out_copies(7, h):
                cp.wait()

    @pl.when(wid >= _ROWS - 7 * _NWORK)
    def _():
        for h in (0, 1):
            for cp in out_copies(6, h):
                cp.wait()

    def edge_drain(ln):
        for b in range(_B):
            for row in (0, 1):
                pltpu.make_async_copy(
                    edg_v.at[pl.ds(row * 1024, ln)],
                    edges_hbm.at[pl.ds(b * 2 * _NE + row * _NE + wid * 1024,
                                       ln)],
                    sem_e).wait()

    @pl.when(wid < _NWORK - 1)
    def _():
        edge_drain(1024)

    @pl.when(wid == _NWORK - 1)
    def _():
        edge_drain(8)


def kernel(x):
    mesh = plsc.VectorSubcoreMesh(core_axis_name="c", subcore_axis_name="s")
    patches_f, pos_f, edges_f = pl.kernel(
        _sc_body,
        out_type=[
            jax.ShapeDtypeStruct((_N, _F // 128 * _B, 128), jnp.float32),
            jax.ShapeDtypeStruct((_B * _N * 2,), jnp.float32),
            jax.ShapeDtypeStruct((_B * 2 * _NE,), jnp.int32),
        ],
        mesh=mesh,
        compiler_params=pltpu.CompilerParams(needs_layout_passes=False),
        scratch_types=[
            pltpu.VMEM((2 * _SLAB,), jnp.float32),
            pltpu.VMEM((2 * _F // 128 * 32, 128), jnp.float32),
            pltpu.VMEM((2048,), jnp.int32),
            pltpu.SemaphoreType.DMA,
            pltpu.SemaphoreType.DMA,
            pltpu.SemaphoreType.DMA,
            pltpu.SemaphoreType.DMA,
        ],
    )(x.reshape(_B, _C, _H // 8, 8, _W // 128, 128)
      .transpose(0, 1, 2, 4, 3, 5).reshape(-1))
    return (jnp.transpose(patches_f.reshape(_N, _F // 128, _B, 128),
                          (2, 0, 1, 3)).reshape(_B, _N, _F),
            pos_f.reshape(_B, _N, 2),
            edges_f.reshape(_B, 2, _NE))


# speedup vs baseline: 129.9626x; 1.0249x over previous
import jax
import jax.numpy as jnp
from jax import lax
from jax.experimental import pallas as pl
from jax.experimental.pallas import tpu as pltpu
from jax.experimental.pallas import tpu_sc as plsc

_B, _C, _H, _W = 4, 3, 512, 512
_P, _S = 16, 8
_NH = (_H - _P) // _S + 1
_NW = (_W - _P) // _S + 1
_N = _NH * _NW
_F = _C * _P * _P
_K = 8
_NWORK = 32
_ROWS = _B * _NH
_NE = _N * _K
_NCH = (_N + 15) // 16
_BIG = 2**31 - 1
_SLAB = _C * _P * _W
_HALF = 32 * _F


def _tree_min(vs):
    while len(vs) > 1:
        nxt = [jnp.minimum(vs[i], vs[i + 1]) for i in range(0, len(vs) - 1, 2)]
        if len(vs) % 2:
            nxt.append(vs[-1])
        vs = nxt
    return vs[0]


def _sc_body(x_hbm, patches_hbm, pos_hbm, edges_hbm, in_v, out_v, edg_v,
             sem_in, sem_out0, sem_out1, sem_e):
    wid = lax.axis_index("s") * 2 + lax.axis_index("c")
    lane = lax.iota(jnp.int32, 16)
    sems_out = (sem_out0, sem_out1)

    def row_bi(t):
        r = jnp.minimum(wid + _NWORK * t, _ROWS - 1)
        return lax.div(r, _NH), lax.rem(r, _NH)

    def valid(t):
        return (wid + _NWORK * t) < _ROWS

    def in_copies(t):
        b, i = row_bi(t)
        buf = jnp.bitwise_and(t, 1)
        return [pltpu.make_async_copy(
            x_hbm.at[pl.ds(((b * _C + c) * (_H // 8) + i) * 4096, _P * _W)],
            in_v.at[pl.ds(buf * _SLAB + c * _P * _W, _P * _W)],
            sem_in) for c in range(_C)]

    def out_copies(t, h):
        b, i = row_bi(t)
        ln = 32 if h == 0 else 31
        return [pltpu.make_async_copy(
            out_v.at[pl.ds((h * 6 + tc) * 32, ln), :],
            patches_hbm.at[pl.ds(i * _NW + h * 32, ln), tc * _B + b, :],
            sems_out[h]) for tc in range(_F // 128)]

    def compute_half(t, h):
        ln = 32 if h == 0 else 31
        base = jnp.bitwise_and(t, 1) * _SLAB

        segs = [(c, kh) for c in range(_C) for kh in range(_P)]

        @plsc.parallel_loop(0, ln, unroll=2)
        def _(jl):
            col = (jl + h * 32) * _S + lane
            cadj = col + jnp.right_shift(col, 7) * 896 + base
            for g in range(0, len(segs), 16):
                grp = segs[g:g + 16]
                vs = [plsc.load_gather(
                    in_v, [cadj + (c * _P * _W + (kh // 8) * 4096
                                   + (kh % 8) * 128)])
                    for c, kh in grp]
                for (c, kh), v in zip(grp, vs):
                    f = c * _P * _P + kh * _P
                    out_v[(h * 6 + f // 128) * 32 + jl,
                          pl.ds(f % 128, _P)] = v

    @pl.when(wid < _NWORK - 1)
    def _():
        for cp in in_copies(0):
            cp.start()

    nch = jnp.minimum(8, _NCH - wid * 8)

    @plsc.parallel_loop(0, nch)
    def _(chl):
        n = (wid * 8 + chl) * 16 + lane
        validn = n < _N
        gi = lax.div(n, _NW)
        gj = lax.rem(n, _NW)
        keys = []
        for di in range(-2, 3):
            ii = gi + di
            vi = (ii >= 0) & (ii < _NH) & validn
            for dj in range(-2, 3):
                if (di, dj) == (0, 0):
                    continue
                jj = gj + dj
                v = vi & (jj >= 0) & (jj < _NW)
                d2 = di * di + dj * dj
                keys.append(jnp.where(v, n + (d2 * 4096 + di * _NW + dj),
                                      _BIG))
        floc = chl * 128 + lane * _K
        for k in range(_K):
            m = _tree_min(keys)
            nbr = jnp.bitwise_and(m, 4095)
            plsc.store_scatter(edg_v, [floc + k], n)
            plsc.store_scatter(edg_v, [1024 + floc + k], nbr)
            if k < _K - 1:
                keys = [jnp.where(kk == m, _BIG, kk) for kk in keys]

    def edge_out(ln):
        for b in range(_B):
            for row in (0, 1):
                pltpu.make_async_copy(
                    edg_v.at[pl.ds(row * 1024, ln)],
                    edges_hbm.at[pl.ds(b * 2 * _NE + row * _NE + wid * 1024,
                                       ln)],
                    sem_e).start()

    @pl.when(wid < _NWORK - 1)
    def _():
        edge_out(1024)

    @pl.when(wid == _NWORK - 1)
    def _():
        edge_out(8)

        @plsc.parallel_loop(0, _NCH)
        def _(ch):
            n = ch * 16 + lane
            validn = n < _N
            nc = jnp.where(validn, n, 0)
            gi = lax.div(nc, _NW).astype(jnp.float32)
            gj = lax.rem(nc, _NW).astype(jnp.float32)
            for b in range(_B):
                plsc.store_scatter(in_v, [b * (2 * _N) + nc * 2], gi,
                                   mask=validn)
                plsc.store_scatter(in_v, [b * (2 * _N) + nc * 2 + 1], gj,
                                   mask=validn)

        pltpu.sync_copy(in_v.at[pl.ds(0, _B * 2 * _N)], pos_hbm)
        for cp in in_copies(0):
            cp.start()

    def tbody(t, carry):
        @pl.when(valid(t))
        def _():
            for cp in in_copies(t):
                cp.wait()

            @pl.when(valid(t + 1))
            def _():
                for cp in in_copies(t + 1):
                    cp.start()

            for h in (0, 1):
                @pl.when(t > 0)
                def _():
                    for cp in out_copies(t - 1, h):
                        cp.wait()

                compute_half(t, h)
                for cp in out_copies(t, h):
                    cp.start()
        return carry

    lax.fori_loop(0, 8, tbody, 0)

    @pl.when(wid < _ROWS - 7 * _NWORK)
    def _():
        for h in (0, 1):
            for cp in out_copies(7, h):
                cp.wait()

    @pl.when(wid >= _ROWS - 7 * _NWORK)
    def _():
        for h in (0, 1):
            for cp in out_copies(6, h):
                cp.wait()

    def edge_drain(ln):
        for b in range(_B):
            for row in (0, 1):
                pltpu.make_async_copy(
                    edg_v.at[pl.ds(row * 1024, ln)],
                    edges_hbm.at[pl.ds(b * 2 * _NE + row * _NE + wid * 1024,
                                       ln)],
                    sem_e).wait()

    @pl.when(wid < _NWORK - 1)
    def _():
        edge_drain(1024)

    @pl.when(wid == _NWORK - 1)
    def _():
        edge_drain(8)


def kernel(x):
    mesh = plsc.VectorSubcoreMesh(core_axis_name="c", subcore_axis_name="s")
    patches_f, pos_f, edges_f = pl.kernel(
        _sc_body,
        out_type=[
            jax.ShapeDtypeStruct((_N, _F // 128 * _B, 128), jnp.float32),
            jax.ShapeDtypeStruct((_B * _N * 2,), jnp.float32),
            jax.ShapeDtypeStruct((_B * 2 * _NE,), jnp.int32),
        ],
        mesh=mesh,
        compiler_params=pltpu.CompilerParams(needs_layout_passes=False),
        scratch_types=[
            pltpu.VMEM((2 * _SLAB,), jnp.float32),
            pltpu.VMEM((2 * _F // 128 * 32, 128), jnp.float32),
            pltpu.VMEM((2048,), jnp.int32),
            pltpu.SemaphoreType.DMA,
            pltpu.SemaphoreType.DMA,
            pltpu.SemaphoreType.DMA,
            pltpu.SemaphoreType.DMA,
        ],
    )(x.reshape(_B, _C, _H // 8, 8, _W // 128, 128)
      .transpose(0, 1, 2, 4, 3, 5).reshape(-1))
    return (jnp.transpose(patches_f.reshape(_N, _F // 128, _B, 128),
                          (2, 0, 1, 3)).reshape(_B, _N, _F),
            pos_f.reshape(_B, _N, 2),
            edges_f.reshape(_B, 2, _NE))


# speedup vs baseline: 131.7286x; 1.0136x over previous
import jax
import jax.numpy as jnp
from jax import lax
from jax.experimental import pallas as pl
from jax.experimental.pallas import tpu as pltpu
from jax.experimental.pallas import tpu_sc as plsc

_B, _C, _H, _W = 4, 3, 512, 512
_P, _S = 16, 8
_NH = (_H - _P) // _S + 1
_NW = (_W - _P) // _S + 1
_N = _NH * _NW
_F = _C * _P * _P
_K = 8
_NWORK = 32
_ROWS = _B * _NH
_NE = _N * _K
_NCH = (_N + 15) // 16
_BIG = 2**31 - 1
_SLAB = _C * _P * _W
_HALF = 32 * _F


def _tree_min(vs):
    while len(vs) > 1:
        nxt = [jnp.minimum(vs[i], vs[i + 1]) for i in range(0, len(vs) - 1, 2)]
        if len(vs) % 2:
            nxt.append(vs[-1])
        vs = nxt
    return vs[0]


def _sc_body(x_hbm, patches_hbm, pos_hbm, edges_hbm, in_v, out_v, edg_v,
             sem_in, sem_out0, sem_out1, sem_e):
    wid = lax.axis_index("s") * 2 + lax.axis_index("c")
    lane = lax.iota(jnp.int32, 16)
    sems_out = (sem_out0, sem_out1)

    def row_bi(t):
        r = jnp.minimum(wid + _NWORK * t, _ROWS - 1)
        return lax.div(r, _NH), lax.rem(r, _NH)

    def valid(t):
        return (wid + _NWORK * t) < _ROWS

    def in_copies(t):
        b, i = row_bi(t)
        buf = jnp.bitwise_and(t, 1)
        return [pltpu.make_async_copy(
            x_hbm.at[pl.ds(((b * _C + c) * (_H // 8) + i) * 4096, _P * _W)],
            in_v.at[pl.ds(buf * _SLAB + c * _P * _W, _P * _W)],
            sem_in) for c in range(_C)]

    def out_copies(t, h):
        b, i = row_bi(t)
        ln = 32 if h == 0 else 31
        return [pltpu.make_async_copy(
            out_v.at[pl.ds((h * 6 + tc) * 32, ln), :],
            patches_hbm.at[pl.ds(i * _NW + h * 32, ln), tc * _B + b, :],
            sems_out[h]) for tc in range(_F // 128)]

    def compute_half(t, h):
        ln = 32 if h == 0 else 31
        base = jnp.bitwise_and(t, 1) * _SLAB

        segs = [(c, kh) for c in range(_C) for kh in range(_P)]

        @plsc.parallel_loop(0, ln, unroll=2)
        def _(jl):
            col = (jl + h * 32) * _S + lane
            cadj = col + jnp.right_shift(col, 7) * 896 + base
            for g in range(0, len(segs), 16):
                grp = segs[g:g + 16]
                vs = [plsc.load_gather(
                    in_v, [cadj + (c * _P * _W + (kh // 8) * 4096
                                   + (kh % 8) * 128)])
                    for c, kh in grp]
                for (c, kh), v in zip(grp, vs):
                    f = c * _P * _P + kh * _P
                    out_v[(h * 6 + f // 128) * 32 + jl,
                          pl.ds(f % 128, _P)] = v

    for cp in in_copies(0):
        cp.start()

    def tbody(t, carry):
        @pl.when(valid(t))
        def _():
            for cp in in_copies(t):
                cp.wait()

            @pl.when(valid(t + 1))
            def _():
                for cp in in_copies(t + 1):
                    cp.start()

            for h in (0, 1):
                @pl.when(t > 0)
                def _():
                    for cp in out_copies(t - 1, h):
                        cp.wait()

                compute_half(t, h)
                for cp in out_copies(t, h):
                    cp.start()
        return carry

    lax.fori_loop(0, 8, tbody, 0)

    nch = jnp.minimum(8, _NCH - wid * 8)

    @plsc.parallel_loop(0, nch)
    def _(chl):
        n = (wid * 8 + chl) * 16 + lane
        validn = n < _N
        gi = lax.div(n, _NW)
        gj = lax.rem(n, _NW)
        keys = []
        for di in range(-2, 3):
            ii = gi + di
            vi = (ii >= 0) & (ii < _NH) & validn
            for dj in range(-2, 3):
                if (di, dj) == (0, 0):
                    continue
                jj = gj + dj
                v = vi & (jj >= 0) & (jj < _NW)
                d2 = di * di + dj * dj
                keys.append(jnp.where(v, n + (d2 * 4096 + di * _NW + dj),
                                      _BIG))
        floc = chl * 128 + lane * _K
        for k in range(_K):
            m = _tree_min(keys)
            nbr = jnp.bitwise_and(m, 4095)
            plsc.store_scatter(edg_v, [floc + k], n)
            plsc.store_scatter(edg_v, [1024 + floc + k], nbr)
            if k < _K - 1:
                keys = [jnp.where(kk == m, _BIG, kk) for kk in keys]

    def edge_out(ln):
        for b in range(_B):
            for row in (0, 1):
                pltpu.make_async_copy(
                    edg_v.at[pl.ds(row * 1024, ln)],
                    edges_hbm.at[pl.ds(b * 2 * _NE + row * _NE + wid * 1024,
                                       ln)],
                    sem_e).start()

    @pl.when(wid < _NWORK - 1)
    def _():
        edge_out(1024)

    @pl.when(wid == _NWORK - 1)
    def _():
        edge_out(8)

        @plsc.parallel_loop(0, _NCH)
        def _(ch):
            n = ch * 16 + lane
            validn = n < _N
            nc = jnp.where(validn, n, 0)
            gi = lax.div(nc, _NW).astype(jnp.float32)
            gj = lax.rem(nc, _NW).astype(jnp.float32)
            for b in range(_B):
                plsc.store_scatter(in_v, [b * (2 * _N) + nc * 2], gi,
                                   mask=validn)
                plsc.store_scatter(in_v, [b * (2 * _N) + nc * 2 + 1], gj,
                                   mask=validn)

        pltpu.sync_copy(in_v.at[pl.ds(0, _B * 2 * _N)], pos_hbm)

    @pl.when(wid < _ROWS - 7 * _NWORK)
    def _():
        for h in (0, 1):
            for cp in out_copies(7, h):
                cp.wait()

    @pl.when(wid >= _ROWS - 7 * _NWORK)
    def _():
        for h in (0, 1):
            for cp in out_copies(6, h):
                cp.wait()

    def edge_drain(ln):
        for b in range(_B):
            for row in (0, 1):
                pltpu.make_async_copy(
                    edg_v.at[pl.ds(row * 1024, ln)],
                    edges_hbm.at[pl.ds(b * 2 * _NE + row * _NE + wid * 1024,
                                       ln)],
                    sem_e).wait()

    @pl.when(wid < _NWORK - 1)
    def _():
        edge_drain(1024)

    @pl.when(wid == _NWORK - 1)
    def _():
        edge_drain(8)


def kernel(x):
    mesh = plsc.VectorSubcoreMesh(core_axis_name="c", subcore_axis_name="s")
    patches_f, pos_f, edges_f = pl.kernel(
        _sc_body,
        out_type=[
            jax.ShapeDtypeStruct((_N, _F // 128 * _B, 128), jnp.float32),
            jax.ShapeDtypeStruct((_B * _N * 2,), jnp.float32),
            jax.ShapeDtypeStruct((_B * 2 * _NE,), jnp.int32),
        ],
        mesh=mesh,
        compiler_params=pltpu.CompilerParams(needs_layout_passes=False),
        scratch_types=[
            pltpu.VMEM((2 * _SLAB,), jnp.float32),
            pltpu.VMEM((2 * _F // 128 * 32, 128), jnp.float32),
            pltpu.VMEM((2048,), jnp.int32),
            pltpu.SemaphoreType.DMA,
            pltpu.SemaphoreType.DMA,
            pltpu.SemaphoreType.DMA,
            pltpu.SemaphoreType.DMA,
        ],
    )(x.reshape(_B, _C, _H // 8, 8, _W // 128, 128)
      .transpose(0, 1, 2, 4, 3, 5).reshape(-1))
    return (jnp.transpose(patches_f.reshape(_N, _F // 128, _B, 128),
                          (2, 0, 1, 3)).reshape(_B, _N, _F),
            pos_f.reshape(_B, _N, 2),
            edges_f.reshape(_B, 2, _NE))
